# Initial kernel scaffold; baseline (speedup 1.0000x reference)
import jax
import jax.numpy as jnp
from jax.experimental import pallas as pl


def kernel(input_pos_tensors, table):
    raise NotImplementedError("write your pallas kernel here")


# speedup vs baseline: 1.9788x; 1.9788x over previous
import functools

import jax
import jax.numpy as jnp
from jax import lax
from jax.experimental import pallas as pl
from jax.experimental.pallas import tpu as pltpu
from jax.experimental.pallas import tpu_sc as plsc

MAX_LEN = 8192
EMBED_DIM = 1024
BATCH = 4
SEQ = 8192

NC = 2
NS = 16
NW = NC * NS

B_TOTAL = BATCH * SEQ
PER_W = B_TOTAL // NW
CHUNK = 32
NCHUNK = PER_W // CHUNK

_mesh = plsc.VectorSubcoreMesh(core_axis_name="c", subcore_axis_name="s")


@functools.partial(
    pl.kernel,
    out_type=jax.ShapeDtypeStruct((B_TOTAL, EMBED_DIM), jnp.float32),
    mesh=_mesh,
    scratch_types=[
        pltpu.VMEM((NCHUNK, CHUNK), jnp.int32),
        pltpu.VMEM((CHUNK, EMBED_DIM), jnp.float32),
        pltpu.SemaphoreType.DMA,
    ],
)
def _gather_kernel(table_hbm, idx_hbm, out_hbm, idx_v, rows_v, gsem):
    wid = lax.axis_index("s") * NC + lax.axis_index("c")
    base = wid * PER_W
    pltpu.sync_copy(idx_hbm.at[wid], idx_v)

    def step(j, carry):
        pltpu.async_copy(table_hbm.at[idx_v.at[j]], rows_v, gsem).wait()
        pltpu.sync_copy(rows_v, out_hbm.at[pl.ds(base + j * CHUNK, CHUNK)])
        return carry

    lax.fori_loop(0, NCHUNK, step, 0)


def kernel(input_pos_tensors, table):
    idx = jnp.reshape(input_pos_tensors.astype(jnp.int32), (NW, NCHUNK, CHUNK))
    out = _gather_kernel(table, idx)
    return jnp.reshape(out, (BATCH, SEQ, EMBED_DIM))


# speedup vs baseline: 2.3768x; 1.2011x over previous
import functools

import jax
import jax.numpy as jnp
from jax import lax
from jax.experimental import pallas as pl
from jax.experimental.pallas import tpu as pltpu
from jax.experimental.pallas import tpu_sc as plsc

MAX_LEN = 8192
EMBED_DIM = 1024
BATCH = 4
SEQ = 8192

NC = 2
NS = 16
NW = NC * NS

B_TOTAL = BATCH * SEQ
PER_W = B_TOTAL // NW
CHUNK = 16
NCHUNK = PER_W // CHUNK
NBUF = 4
NG = NCHUNK // NBUF

_mesh = plsc.VectorSubcoreMesh(core_axis_name="c", subcore_axis_name="s")


@functools.partial(
    pl.kernel,
    out_type=jax.ShapeDtypeStruct((B_TOTAL, EMBED_DIM), jnp.float32),
    mesh=_mesh,
    scratch_types=[
        pltpu.VMEM((NCHUNK, CHUNK), jnp.int32),
        [pltpu.VMEM((CHUNK, EMBED_DIM), jnp.float32) for _ in range(NBUF)],
        [pltpu.SemaphoreType.DMA for _ in range(NBUF)],
        [pltpu.SemaphoreType.DMA for _ in range(NBUF)],
    ],
)
def _gather_kernel(table_hbm, idx_hbm, out_hbm, idx_v, rows, gsem, wsem):
    wid = lax.axis_index("s") * NC + lax.axis_index("c")
    base = wid * PER_W
    pltpu.sync_copy(idx_hbm.at[wid], idx_v)

    def start_gather(j, b):
        pltpu.async_copy(table_hbm.at[idx_v.at[j]], rows[b], gsem[b])

    def wait_gather(b):
        pltpu.make_async_copy(table_hbm.at[pl.ds(0, CHUNK)], rows[b],
                              gsem[b]).wait()

    def start_write(j, b):
        pltpu.async_copy(rows[b], out_hbm.at[pl.ds(base + j * CHUNK, CHUNK)],
                         wsem[b])

    def wait_write(b):
        pltpu.make_async_copy(rows[b], out_hbm.at[pl.ds(base, CHUNK)],
                              wsem[b]).wait()

    for k in range(NBUF - 1):
        start_gather(k, k)

    def group(g, carry):
        for k in range(NBUF):
            j = g * NBUF + k
            b = k
            bn = (k - 1) % NBUF
            wait_gather(b)
            start_write(j, b)
            jn = j + NBUF - 1
            if k == 0:
                @pl.when(j >= 1)
                def _():
                    wait_write(bn)
            else:
                wait_write(bn)

            @pl.when(jn < NCHUNK)
            def _():
                start_gather(jn, bn)
        return carry

    lax.fori_loop(0, NG, group, 0)
    wait_write((NCHUNK - 1) % NBUF)


def kernel(input_pos_tensors, table):
    idx = jnp.reshape(input_pos_tensors.astype(jnp.int32), (NW, NCHUNK, CHUNK))
    out = _gather_kernel(table, idx)
    return jnp.reshape(out, (BATCH, SEQ, EMBED_DIM))


# speedup vs baseline: 2.3784x; 1.0007x over previous
import functools

import jax
import jax.numpy as jnp
from jax import lax
from jax.experimental import pallas as pl
from jax.experimental.pallas import tpu as pltpu
from jax.experimental.pallas import tpu_sc as plsc

MAX_LEN = 8192
EMBED_DIM = 1024
BATCH = 4
SEQ = 8192

NC = 2
NS = 16
NW = NC * NS

B_TOTAL = BATCH * SEQ
PER_W = B_TOTAL // NW
CHUNK = 16
NCHUNK = PER_W // CHUNK
NBUF = 4
AHEAD = 2
NG = NCHUNK // NBUF

_mesh = plsc.VectorSubcoreMesh(core_axis_name="c", subcore_axis_name="s")


@functools.partial(
    pl.kernel,
    out_type=jax.ShapeDtypeStruct((B_TOTAL, EMBED_DIM), jnp.float32),
    mesh=_mesh,
    scratch_types=[
        pltpu.VMEM((NCHUNK, CHUNK), jnp.int32),
        [pltpu.VMEM((CHUNK, EMBED_DIM), jnp.float32) for _ in range(NBUF)],
        [pltpu.SemaphoreType.DMA for _ in range(NBUF)],
        [pltpu.SemaphoreType.DMA for _ in range(NBUF)],
    ],
)
def _gather_kernel(table_hbm, idx_hbm, out_hbm, idx_v, rows, gsem, wsem):
    wid = lax.axis_index("s") * NC + lax.axis_index("c")
    base = wid * PER_W
    pltpu.sync_copy(idx_hbm.at[wid], idx_v)

    def start_gather(j, b):
        pltpu.async_copy(table_hbm.at[idx_v.at[j]], rows[b], gsem[b])

    def wait_gather(b):
        pltpu.make_async_copy(table_hbm.at[pl.ds(0, CHUNK)], rows[b],
                              gsem[b]).wait()

    def start_write(j, b):
        pltpu.async_copy(rows[b], out_hbm.at[pl.ds(base + j * CHUNK, CHUNK)],
                         wsem[b])

    def wait_write(b):
        pltpu.make_async_copy(rows[b], out_hbm.at[pl.ds(base, CHUNK)],
                              wsem[b]).wait()

    for k in range(AHEAD):
        start_gather(k, k)

    def group(g, carry):
        for k in range(NBUF):
            j = g * NBUF + k
            b = k
            bn = (k + AHEAD) % NBUF
            wait_gather(b)
            start_write(j, b)
            jn = j + AHEAD
            if k < NBUF - AHEAD:
                @pl.when(j >= NBUF - AHEAD)
                def _():
                    wait_write(bn)
            else:
                wait_write(bn)

            @pl.when(jn < NCHUNK)
            def _():
                start_gather(jn, bn)
        return carry

    lax.fori_loop(0, NG, group, 0)
    for j in range(NCHUNK - (NBUF - AHEAD), NCHUNK):
        wait_write(j % NBUF)


def kernel(input_pos_tensors, table):
    idx = jnp.reshape(input_pos_tensors.astype(jnp.int32), (NW, NCHUNK, CHUNK))
    out = _gather_kernel(table, idx)
    return jnp.reshape(out, (BATCH, SEQ, EMBED_DIM))
